# Initial kernel scaffold; baseline (speedup 1.0000x reference)
import jax
import jax.numpy as jnp
from jax.experimental import pallas as pl


def kernel(x, edge_index, batch, target, W1, a_src1, a_dst1, b1, W2, a_src2, a_dst2, b2, W3, a_src3, a_dst3, b3, Wg1, bg1, Wf1, bf1, Wf2, bf2, Wo, bo):
    raise NotImplementedError("write your pallas kernel here")


# speedup vs baseline: 1.0061x; 1.0061x over previous
import functools

import jax
import jax.numpy as jnp
from jax.experimental import pallas as pl
from jax.experimental.pallas import tpu as pltpu


def _ceil_to(v, m):
    return (v + m - 1) // m * m


def _mm_kernel(x_ref, w_ref, o_ref):
    o_ref[...] = jnp.dot(x_ref[...], w_ref[...],
                         preferred_element_type=jnp.float32)


def _matmul(x, w, bn=256, bm=256):
    n, k = x.shape
    k2, m = w.shape
    assert k == k2
    npad, mpad = _ceil_to(n, bn), _ceil_to(m, bm)
    kpad = _ceil_to(k, 128)
    if npad != n or kpad != k:
        x = jnp.pad(x, ((0, npad - n), (0, kpad - k)))
    if mpad != m or kpad != k:
        w = jnp.pad(w, ((0, kpad - k), (0, mpad - m)))
    out = pl.pallas_call(
        _mm_kernel,
        grid=(npad // bn, mpad // bm),
        in_specs=[
            pl.BlockSpec((bn, kpad), lambda i, j: (i, 0)),
            pl.BlockSpec((kpad, bm), lambda i, j: (0, j)),
        ],
        out_specs=pl.BlockSpec((bn, bm), lambda i, j: (i, j)),
        out_shape=jax.ShapeDtypeStruct((npad, mpad), jnp.float32),
    )(x, w)
    return out[:n, :m]


def _gat_layer(x, src, dst, W, a_src, a_dst, b, n):
    H, C = a_src.shape
    h = _matmul(x, W).reshape(n, H, C)
    al_s = (h * a_src[None, :, :]).sum(-1)
    al_d = (h * a_dst[None, :, :]).sum(-1)
    alpha = al_s[src] + al_d[dst]
    alpha = jnp.where(alpha > 0, alpha, 0.2 * alpha)
    ex = jnp.exp(alpha)
    den = jax.ops.segment_sum(ex, dst, num_segments=n)
    coef = ex / den[dst]
    msg = h[src] * coef[:, :, None]
    out = jax.ops.segment_sum(msg, dst, num_segments=n)
    return out.reshape(n, H * C) + b


def kernel(x, edge_index, batch, target, W1, a_src1, a_dst1, b1, W2, a_src2,
           a_dst2, b2, W3, a_src3, a_dst3, b3, Wg1, bg1, Wf1, bf1, Wf2, bf2,
           Wo, bo):
    n = x.shape[0]
    loop = jnp.arange(n, dtype=edge_index.dtype)
    src = jnp.concatenate([edge_index[0], loop])
    dst = jnp.concatenate([edge_index[1], loop])
    h = jax.nn.relu(_gat_layer(x, src, dst, W1, a_src1, a_dst1, b1, n))
    h = jax.nn.relu(_gat_layer(h, src, dst, W2, a_src2, a_dst2, b2, n))
    h = jax.nn.relu(_gat_layer(h, src, dst, W3, a_src3, a_dst3, b3, n))
    g = jax.ops.segment_max(h, batch, num_segments=64)
    g = jax.nn.relu(_matmul(g, Wg1, bn=64, bm=128) + bg1)
    g = jax.nn.relu(_matmul(g, Wf1, bn=64, bm=256) + bf1)
    g = jax.nn.relu(_matmul(g, Wf2, bn=64, bm=256) + bf2)
    out = g @ Wo + bo
    return out
